# baseline (speedup 1.0000x reference)
import functools

import jax
import jax.numpy as jnp
from jax import lax
from jax.experimental import pallas as pl
from jax.experimental.pallas import tpu as pltpu
from jax.experimental.pallas import tpu_sc as plsc

N = 10000
E = 160000
R = 65
H = 128
MH = 64
NG = 128

NC = 2
NS = 16
NW = NC * NS

BB = 256
NBE = E // BB
EG = 163840
CWE = EG // (NW * 128)

EP = 184320
NBLK = EP // BB
CW = EP // (NW * 128)
SLOTS_W = CW * 128

NPAD = 10240
NPT = NPAD // NS
BQ = 1000
NBQ = N // BQ

BIGKEY = 1 << 30


def _mesh():
    return plsc.VectorSubcoreMesh(core_axis_name="c", subcore_axis_name="s")


def _gather_sc(table, idx3, cwx):
    @functools.partial(
        pl.kernel,
        mesh=_mesh(),
        out_type=jax.ShapeDtypeStruct((NW * cwx * 128, H), jnp.float32),
        scratch_types=[
            pltpu.VMEM((cwx, 128), jnp.int32),
            pltpu.VMEM((128, H), jnp.float32),
            pltpu.VMEM((128, H), jnp.float32),
            pltpu.SemaphoreType.DMA,
            pltpu.SemaphoreType.DMA,
        ],
    )
    def k(tab_hbm, idx_hbm, out_hbm, idxv, rows0, rows1, sem0, sem1):
        c = lax.axis_index("c")
        s = lax.axis_index("s")
        w = s * NC + c
        base = w * cwx * 128
        pltpu.sync_copy(idx_hbm.at[w], idxv)
        bufs = (rows0, rows1)
        sems = (sem0, sem1)
        cps = [None, None]
        cps[0] = pltpu.async_copy(tab_hbm.at[idxv.at[0]], rows0, sem0)
        for j in range(cwx):
            cps[j % 2].wait()
            if j + 1 < cwx:
                cps[(j + 1) % 2] = pltpu.async_copy(
                    tab_hbm.at[idxv.at[j + 1]], bufs[(j + 1) % 2], sems[(j + 1) % 2])
            pltpu.sync_copy(bufs[j % 2], out_hbm.at[pl.ds(base + j * 128, 128)])

    return k(table, idx3)


def _scatter_sc(msg, dstq3, zn):
    @functools.partial(
        pl.kernel,
        mesh=_mesh(),
        out_type=jax.ShapeDtypeStruct((NW, NPT, H), jnp.float32),
        scratch_types=[
            pltpu.VMEM((CW, 128), jnp.int32),
            pltpu.VMEM((128, H), jnp.float32),
            pltpu.VMEM((128, H), jnp.float32),
            pltpu.VMEM_SHARED((NPAD, H), jnp.float32),
            pltpu.SemaphoreType.DMA,
            pltpu.SemaphoreType.DMA,
        ],
    )
    def k(msg_hbm, dstq_hbm, zn_hbm, part_hbm, idxv, rows0, rows1, acc, sem0, sem1):
        c = lax.axis_index("c")
        s = lax.axis_index("s")
        w = s * NC + c
        base = w * SLOTS_W
        pltpu.sync_copy(zn_hbm, rows0)
        for t in range(NPT // 128):
            pltpu.sync_copy(rows0, acc.at[pl.ds(s * NPT + t * 128, 128)])
        pltpu.sync_copy(dstq_hbm.at[w], idxv)
        plsc.subcore_barrier()
        bufs = (rows0, rows1)
        sems = (sem0, sem1)
        cps = [None, None]
        cps[0] = pltpu.async_copy(msg_hbm.at[pl.ds(base, 128)], rows0, sem0)
        for j in range(CW):
            cps[j % 2].wait()
            if j + 1 < CW:
                cps[(j + 1) % 2] = pltpu.async_copy(
                    msg_hbm.at[pl.ds(base + (j + 1) * 128, 128)],
                    bufs[(j + 1) % 2], sems[(j + 1) % 2])
            pltpu.sync_copy(bufs[j % 2], acc.at[idxv.at[j]], add=True)
        plsc.subcore_barrier()
        for t in range(NPT // 128):
            pltpu.sync_copy(acc.at[pl.ds(s * NPT + t * 128, 128)], rows0)
            pltpu.sync_copy(rows0, part_hbm.at[c * NS + s].at[pl.ds(t * 128, 128)])

    return k(msg, dstq3, zn)


def _binsum_tc(xs, lbcol, cont_in, lbmax):
    def body(cont_ref, lbm_ref, xs_ref, lb_ref, out_ref, carry):
        i = pl.program_id(0)
        oh = (lb_ref[...] == lax.broadcasted_iota(jnp.int32, (BB, BB), 1)
              ).astype(jnp.float32)
        p = lax.dot_general(oh, xs_ref[...], (((0,), (0,)), ((), ())),
                            precision=lax.Precision.HIGHEST,
                            preferred_element_type=jnp.float32)
        cont = (cont_ref[i] == 1) & (i > 0)
        row0 = (lax.broadcasted_iota(jnp.int32, (BB, 1), 0) == 0
                ).astype(jnp.float32)
        cval = jnp.where(cont, carry[...], jnp.zeros((1, H), jnp.float32))
        p = p + row0 * cval
        lm = lbm_ref[i]
        msk = (lax.broadcasted_iota(jnp.int32, (BB, 1), 0) == lm
               ).astype(jnp.float32)
        carry[...] = jnp.sum(p * msk, axis=0, keepdims=True)
        out_ref[...] = p

    grid_spec = pltpu.PrefetchScalarGridSpec(
        num_scalar_prefetch=2,
        grid=(NBE,),
        in_specs=[
            pl.BlockSpec((BB, H), lambda i, c, l: (i, 0)),
            pl.BlockSpec((BB, 1), lambda i, c, l: (i, 0)),
        ],
        out_specs=pl.BlockSpec((BB, H), lambda i, c, l: (i, 0)),
        scratch_shapes=[pltpu.VMEM((1, H), jnp.float32)],
    )
    return pl.pallas_call(
        body, grid_spec=grid_spec,
        out_shape=jax.ShapeDtypeStruct((E, H), jnp.float32),
    )(cont_in, lbmax, xs[:E], lbcol)


def _relmm_tc(binsum, cntq2, rw, blk2rel):
    def body(b2r, bs_ref, cnt_ref, w_ref, out_ref):
        mean = bs_ref[...] / cnt_ref[...]
        out_ref[...] = jnp.dot(mean, w_ref[0], preferred_element_type=jnp.float32)

    grid_spec = pltpu.PrefetchScalarGridSpec(
        num_scalar_prefetch=1,
        grid=(NBLK,),
        in_specs=[
            pl.BlockSpec((BB, H), lambda i, b2r: (i, 0)),
            pl.BlockSpec((BB, 1), lambda i, b2r: (i, 0)),
            pl.BlockSpec((1, H, H), lambda i, b2r: (b2r[i], 0, 0)),
        ],
        out_specs=pl.BlockSpec((BB, H), lambda i, b2r: (i, 0)),
    )
    return pl.pallas_call(
        body, grid_spec=grid_spec,
        out_shape=jax.ShapeDtypeStruct((EP, H), jnp.float32),
    )(blk2rel, binsum, cntq2, rw)


def _combine_tc(p0, p1, h, root, b2):
    def body(p0_ref, p1_ref, h_ref, r_ref, b_ref, out_ref):
        ein = p0_ref[...] + p1_ref[...]
        acc = jnp.dot(h_ref[...], r_ref[...], preferred_element_type=jnp.float32)
        out_ref[...] = jnp.maximum((ein + acc) + b_ref[...], 0.0)

    return pl.pallas_call(
        body,
        grid=(NBQ,),
        in_specs=[
            pl.BlockSpec((BQ, H), lambda i: (i, 0)),
            pl.BlockSpec((BQ, H), lambda i: (i, 0)),
            pl.BlockSpec((BQ, H), lambda i: (i, 0)),
            pl.BlockSpec((H, H), lambda i: (0, 0)),
            pl.BlockSpec((1, H), lambda i: (0, 0)),
        ],
        out_specs=pl.BlockSpec((BQ, H), lambda i: (i, 0)),
        out_shape=jax.ShapeDtypeStruct((N, H), jnp.float32),
    )(p0, p1, h, root, b2)


def _head_tc(h, batch2, ws_w, ws_b, mw0, mb0, g0, be0, mw1, mb1, g1, be1,
             mw2, mb2):
    def bn(z, gamma, beta):
        mu = jnp.mean(z, axis=0, keepdims=True)
        var = jnp.mean((z - mu) * (z - mu), axis=0, keepdims=True)
        return (z - mu) / jnp.sqrt(var + 1e-5) * gamma + beta

    def body(h_ref, b_ref, wsw_ref, wsb_ref, mw0_ref, mb0_ref, g0_ref, be0_ref,
             mw1_ref, mb1_ref, g1_ref, be1_ref, mw2_ref, mb2_ref, out_ref, gacc):
        i = pl.program_id(0)

        @pl.when(i == 0)
        def _():
            gacc[...] = jnp.zeros((NG, H), jnp.float32)

        hv = h_ref[...]
        wv = jax.nn.sigmoid(
            jnp.dot(hv, wsw_ref[...], preferred_element_type=jnp.float32)
            + wsb_ref[0, 0])
        wh = wv * hv
        onehot = (b_ref[...] == lax.broadcasted_iota(jnp.int32, (BQ, NG), 1)
                  ).astype(jnp.float32)
        gacc[...] += lax.dot_general(onehot, wh, (((0,), (0,)), ((), ())),
                                     precision=lax.Precision.HIGHEST,
                                     preferred_element_type=jnp.float32)

        @pl.when(i == NBQ - 1)
        def _():
            g = gacc[...]
            z = jnp.dot(g, mw0_ref[...], preferred_element_type=jnp.float32)
            z = jnp.maximum(bn(z + mb0_ref[...], g0_ref[...], be0_ref[...]), 0.0)
            z = jnp.dot(z, mw1_ref[...], preferred_element_type=jnp.float32)
            z = jnp.maximum(bn(z + mb1_ref[...], g1_ref[...], be1_ref[...]), 0.0)
            out_ref[...] = jnp.dot(z, mw2_ref[...],
                                   preferred_element_type=jnp.float32) + mb2_ref[0, 0]

    def full(shape):
        return pl.BlockSpec(shape, lambda i: tuple(0 for _ in shape))

    return pl.pallas_call(
        body,
        grid=(NBQ,),
        in_specs=[
            pl.BlockSpec((BQ, H), lambda i: (i, 0)),
            pl.BlockSpec((BQ, 1), lambda i: (i, 0)),
            full((H, 1)), full((1, 1)),
            full((H, MH)), full((1, MH)), full((1, MH)), full((1, MH)),
            full((MH, MH)), full((1, MH)), full((1, MH)), full((1, MH)),
            full((MH, 1)), full((1, 1)),
        ],
        out_specs=pl.BlockSpec((NG, 1), lambda i: (0, 0)),
        out_shape=jax.ShapeDtypeStruct((NG, 1), jnp.float32),
        scratch_shapes=[pltpu.VMEM((NG, H), jnp.float32)],
    )(h, batch2, ws_w, ws_b, mw0, mb0, g0, be0, mw1, mb1, g1, be1, mw2, mb2)


def kernel(x, edge_index, edge_type, batch,
           rw0, rr0, rb0, rw1, rr1, rb1, rw2, rr2, rb2,
           ws_w, ws_b, mw0, mb0, g0, be0, mw1, mb1, g1, be1, mw2, mb2):
    i32 = jnp.int32
    src = edge_index[0].astype(i32)
    dst = edge_index[1].astype(i32)
    et = edge_type.reshape(-1).astype(i32)

    key = et * N + dst
    key_s, order = lax.sort_key_val(key, jnp.arange(E, dtype=i32))
    src_s = jnp.take(src, order)
    newbin = jnp.concatenate([jnp.ones((1,), i32),
                              (key_s[1:] != key_s[:-1]).astype(i32)])
    bidx = jnp.cumsum(newbin).astype(i32) - 1
    barr = jnp.arange(E, dtype=i32)
    first_e = jnp.searchsorted(bidx, barr, side="left").astype(i32)
    last_e = jnp.searchsorted(bidx, barr, side="right").astype(i32) - 1
    validb = first_e < E
    c_b = jnp.where(validb, last_e - first_e + 1, 1)
    key_b = jnp.take(key_s, jnp.clip(first_e, 0, E - 1))
    key_bm = jnp.where(validb, key_b, BIGKEY)
    bstart = jnp.searchsorted(key_bm, jnp.arange(R + 1, dtype=i32) * N,
                              side="left").astype(i32)
    brelcnt = bstart[1:] - bstart[:-1]
    nblkb = (brelcnt + BB - 1) // BB
    blk_end = jnp.cumsum(nblkb).astype(i32)
    blk2rel = jnp.minimum(
        jnp.searchsorted(blk_end, jnp.arange(NBLK, dtype=i32), side="right"),
        R - 1).astype(i32)
    pad_off = (blk_end - nblkb) * BB

    qq = jnp.arange(EP, dtype=i32)
    r_q = jnp.take(blk2rel, qq // BB)
    pos = qq - jnp.take(pad_off, r_q)
    bc = jnp.take(brelcnt, r_q)
    validq = (pos >= 0) & (pos < bc)
    b_q = jnp.clip(jnp.take(bstart[:-1], r_q) + jnp.clip(pos, 0, E - 1), 0, E - 1)
    dstq = jnp.where(validq, jnp.take(key_b, b_q) % N, 0)
    cntq = jnp.where(validq, jnp.take(c_b, b_q).astype(jnp.float32), jnp.inf)

    startbin = jnp.take(bidx, jnp.arange(NBE, dtype=i32) * BB)
    lbcol = (bidx - jnp.take(startbin, barr // BB)).reshape(E, 1)
    cont_in = jnp.concatenate([
        jnp.zeros((1,), i32),
        (jnp.take(bidx, jnp.arange(1, NBE, dtype=i32) * BB)
         == jnp.take(bidx, jnp.arange(1, NBE, dtype=i32) * BB - 1)).astype(i32)])
    lbmax = (jnp.take(bidx, jnp.arange(NBE, dtype=i32) * BB + (BB - 1))
             - startbin).astype(i32)
    le_q = jnp.take(last_e, b_q)
    i_end = le_q // BB
    binpos = jnp.where(validq,
                       i_end * BB + jnp.take(bidx, le_q)
                       - jnp.take(startbin, i_end), 0)

    src_pad = jnp.where(jnp.arange(EG, dtype=i32) < E,
                        jnp.take(src_s, jnp.clip(jnp.arange(EG, dtype=i32),
                                                 0, E - 1)), 0)
    srcp3 = src_pad.reshape(NW, CWE, 128)
    binpos3 = binpos.reshape(NW, CW, 128)
    dstq3 = dstq.reshape(NW, CW, 128)
    cntq2 = cntq.reshape(EP, 1)
    zn = jnp.zeros((128, H), jnp.float32)

    h = x
    for rw, rr, rb in ((rw0, rr0, rb0), (rw1, rr1, rb1), (rw2, rr2, rb2)):
        xs = _gather_sc(h, srcp3, CWE)
        bsum_ends = _binsum_tc(xs, lbcol, cont_in, lbmax)
        bsum = _gather_sc(bsum_ends, binpos3, CW)
        msg = _relmm_tc(bsum, cntq2, rw, blk2rel)
        part = _scatter_sc(msg, dstq3, zn).reshape(NC, NPAD, H)
        h = _combine_tc(part[0, :N], part[1, :N], h, rr, rb.reshape(1, H))

    return _head_tc(h, batch.reshape(N, 1).astype(i32),
                    ws_w, ws_b.reshape(1, 1), mw0, mb0.reshape(1, MH),
                    g0.reshape(1, MH), be0.reshape(1, MH), mw1,
                    mb1.reshape(1, MH), g1.reshape(1, MH), be1.reshape(1, MH),
                    mw2, mb2.reshape(1, 1))


# speedup vs baseline: 1.4670x; 1.4670x over previous
import functools

import jax
import jax.numpy as jnp
from jax import lax
from jax.experimental import pallas as pl
from jax.experimental.pallas import tpu as pltpu
from jax.experimental.pallas import tpu_sc as plsc

N = 10000
E = 160000
R = 65
H = 128
MH = 64
NG = 128

NC = 2
NS = 16
NW = NC * NS

BB = 256
NBE = E // BB
EG = 163840
CWE = EG // (NW * 128)

EP = 184320
NBLK = EP // BB
CW = EP // (NW * 128)
SLOTS_W = CW * 128

NPAD = 10240
NPT = NPAD // NS
BQ = 1000
NBQ = N // BQ

BIGKEY = 1 << 30


def _mesh():
    return plsc.VectorSubcoreMesh(core_axis_name="c", subcore_axis_name="s")


NBUF = 6
NBUF_S = 2


def _gather_sc(table, idx3, cwx):
    @functools.partial(
        pl.kernel,
        mesh=_mesh(),
        out_type=jax.ShapeDtypeStruct((NW * cwx * 128, H), jnp.float32),
        scratch_types=(
            [pltpu.VMEM((cwx, 128), jnp.int32)]
            + [pltpu.VMEM((128, H), jnp.float32) for _ in range(NBUF)]
            + [pltpu.SemaphoreType.DMA for _ in range(2 * NBUF)]
        ),
    )
    def k(tab_hbm, idx_hbm, out_hbm, idxv, *bs):
        bufs = bs[:NBUF]
        gsem = bs[NBUF:2 * NBUF]
        wsem = bs[2 * NBUF:]
        c = lax.axis_index("c")
        s = lax.axis_index("s")
        w = s * NC + c
        base = w * cwx * 128
        pltpu.sync_copy(idx_hbm.at[w], idxv)
        gcp = [None] * cwx
        wcp = [None] * cwx
        for j in range(min(NBUF, cwx)):
            gcp[j] = pltpu.async_copy(tab_hbm.at[idxv.at[j]], bufs[j % NBUF],
                                      gsem[j % NBUF])
        for j in range(cwx):
            gcp[j].wait()
            wcp[j] = pltpu.async_copy(bufs[j % NBUF],
                                      out_hbm.at[pl.ds(base + j * 128, 128)],
                                      wsem[j % NBUF])
            nj = j + NBUF
            if nj < cwx:
                wcp[j].wait()
                gcp[nj] = pltpu.async_copy(tab_hbm.at[idxv.at[nj]],
                                           bufs[nj % NBUF], gsem[nj % NBUF])
        for j in range(max(0, cwx - NBUF), cwx):
            wcp[j].wait()

    return k(table, idx3)


def _scatter_sc(msg, dstq3, zn):
    @functools.partial(
        pl.kernel,
        mesh=_mesh(),
        out_type=jax.ShapeDtypeStruct((NW, NPT, H), jnp.float32),
        scratch_types=(
            [pltpu.VMEM((CW, 128), jnp.int32)]
            + [pltpu.VMEM((128, H), jnp.float32) for _ in range(NBUF_S)]
            + [pltpu.VMEM_SHARED((NPAD, H), jnp.float32)]
            + [pltpu.SemaphoreType.DMA for _ in range(2 * NBUF_S)]
        ),
    )
    def k(msg_hbm, dstq_hbm, zn_hbm, part_hbm, idxv, *bs):
        bufs = bs[:NBUF_S]
        acc = bs[NBUF_S]
        lsem = bs[NBUF_S + 1:NBUF_S + 1 + NBUF_S]
        ssem = bs[NBUF_S + 1 + NBUF_S:]
        c = lax.axis_index("c")
        s = lax.axis_index("s")
        w = s * NC + c
        base = w * SLOTS_W
        pltpu.sync_copy(zn_hbm, bufs[0])
        for t in range(NPT // 128):
            pltpu.sync_copy(bufs[0], acc.at[pl.ds(s * NPT + t * 128, 128)])
        pltpu.sync_copy(dstq_hbm.at[w], idxv)
        plsc.subcore_barrier()
        lcp = [None] * CW
        scp = [None] * CW
        for j in range(min(NBUF_S, CW)):
            lcp[j] = pltpu.async_copy(msg_hbm.at[pl.ds(base + j * 128, 128)],
                                      bufs[j % NBUF_S], lsem[j % NBUF_S])
        for j in range(CW):
            lcp[j].wait()
            scp[j] = pltpu.async_copy(bufs[j % NBUF_S], acc.at[idxv.at[j]],
                                      ssem[j % NBUF_S], add=True)
            nj = j + NBUF_S
            if nj < CW:
                scp[j].wait()
                lcp[nj] = pltpu.async_copy(
                    msg_hbm.at[pl.ds(base + nj * 128, 128)],
                    bufs[nj % NBUF_S], lsem[nj % NBUF_S])
        for j in range(max(0, CW - NBUF_S), CW):
            scp[j].wait()
        plsc.subcore_barrier()
        for t in range(NPT // 128):
            pltpu.sync_copy(acc.at[pl.ds(s * NPT + t * 128, 128)], bufs[0])
            pltpu.sync_copy(bufs[0], part_hbm.at[c * NS + s].at[pl.ds(t * 128, 128)])

    return k(msg, dstq3, zn)


def _binsum_tc(xs, lbcol, cont_in, lbmax):
    def body(cont_ref, lbm_ref, xs_ref, lb_ref, out_ref, carry):
        i = pl.program_id(0)
        oh = (lb_ref[...] == lax.broadcasted_iota(jnp.int32, (BB, BB), 1)
              ).astype(jnp.float32)
        p = lax.dot_general(oh, xs_ref[...], (((0,), (0,)), ((), ())),
                            precision=lax.Precision.HIGHEST,
                            preferred_element_type=jnp.float32)
        cont = (cont_ref[i] == 1) & (i > 0)
        row0 = (lax.broadcasted_iota(jnp.int32, (BB, 1), 0) == 0
                ).astype(jnp.float32)
        cval = jnp.where(cont, carry[...], jnp.zeros((1, H), jnp.float32))
        p = p + row0 * cval
        lm = lbm_ref[i]
        msk = (lax.broadcasted_iota(jnp.int32, (BB, 1), 0) == lm
               ).astype(jnp.float32)
        carry[...] = jnp.sum(p * msk, axis=0, keepdims=True)
        out_ref[...] = p

    grid_spec = pltpu.PrefetchScalarGridSpec(
        num_scalar_prefetch=2,
        grid=(NBE,),
        in_specs=[
            pl.BlockSpec((BB, H), lambda i, c, l: (i, 0)),
            pl.BlockSpec((BB, 1), lambda i, c, l: (i, 0)),
        ],
        out_specs=pl.BlockSpec((BB, H), lambda i, c, l: (i, 0)),
        scratch_shapes=[pltpu.VMEM((1, H), jnp.float32)],
    )
    return pl.pallas_call(
        body, grid_spec=grid_spec,
        out_shape=jax.ShapeDtypeStruct((E, H), jnp.float32),
    )(cont_in, lbmax, xs[:E], lbcol)


def _relmm_tc(binsum, cntq2, rw, blk2rel):
    def body(b2r, bs_ref, cnt_ref, w_ref, out_ref):
        mean = bs_ref[...] / cnt_ref[...]
        out_ref[...] = jnp.dot(mean, w_ref[0], preferred_element_type=jnp.float32)

    grid_spec = pltpu.PrefetchScalarGridSpec(
        num_scalar_prefetch=1,
        grid=(NBLK,),
        in_specs=[
            pl.BlockSpec((BB, H), lambda i, b2r: (i, 0)),
            pl.BlockSpec((BB, 1), lambda i, b2r: (i, 0)),
            pl.BlockSpec((1, H, H), lambda i, b2r: (b2r[i], 0, 0)),
        ],
        out_specs=pl.BlockSpec((BB, H), lambda i, b2r: (i, 0)),
    )
    return pl.pallas_call(
        body, grid_spec=grid_spec,
        out_shape=jax.ShapeDtypeStruct((EP, H), jnp.float32),
    )(blk2rel, binsum, cntq2, rw)


def _combine_tc(p0, p1, h, root, b2):
    def body(p0_ref, p1_ref, h_ref, r_ref, b_ref, out_ref):
        ein = p0_ref[...] + p1_ref[...]
        acc = jnp.dot(h_ref[...], r_ref[...], preferred_element_type=jnp.float32)
        out_ref[...] = jnp.maximum((ein + acc) + b_ref[...], 0.0)

    return pl.pallas_call(
        body,
        grid=(NBQ,),
        in_specs=[
            pl.BlockSpec((BQ, H), lambda i: (i, 0)),
            pl.BlockSpec((BQ, H), lambda i: (i, 0)),
            pl.BlockSpec((BQ, H), lambda i: (i, 0)),
            pl.BlockSpec((H, H), lambda i: (0, 0)),
            pl.BlockSpec((1, H), lambda i: (0, 0)),
        ],
        out_specs=pl.BlockSpec((BQ, H), lambda i: (i, 0)),
        out_shape=jax.ShapeDtypeStruct((N, H), jnp.float32),
    )(p0, p1, h, root, b2)


def _head_tc(h, batch2, ws_w, ws_b, mw0, mb0, g0, be0, mw1, mb1, g1, be1,
             mw2, mb2):
    def bn(z, gamma, beta):
        mu = jnp.mean(z, axis=0, keepdims=True)
        var = jnp.mean((z - mu) * (z - mu), axis=0, keepdims=True)
        return (z - mu) / jnp.sqrt(var + 1e-5) * gamma + beta

    def body(h_ref, b_ref, wsw_ref, wsb_ref, mw0_ref, mb0_ref, g0_ref, be0_ref,
             mw1_ref, mb1_ref, g1_ref, be1_ref, mw2_ref, mb2_ref, out_ref, gacc):
        i = pl.program_id(0)

        @pl.when(i == 0)
        def _():
            gacc[...] = jnp.zeros((NG, H), jnp.float32)

        hv = h_ref[...]
        wv = jax.nn.sigmoid(
            jnp.dot(hv, wsw_ref[...], preferred_element_type=jnp.float32)
            + wsb_ref[0, 0])
        wh = wv * hv
        onehot = (b_ref[...] == lax.broadcasted_iota(jnp.int32, (BQ, NG), 1)
                  ).astype(jnp.float32)
        gacc[...] += lax.dot_general(onehot, wh, (((0,), (0,)), ((), ())),
                                     precision=lax.Precision.HIGHEST,
                                     preferred_element_type=jnp.float32)

        @pl.when(i == NBQ - 1)
        def _():
            g = gacc[...]
            z = jnp.dot(g, mw0_ref[...], preferred_element_type=jnp.float32)
            z = jnp.maximum(bn(z + mb0_ref[...], g0_ref[...], be0_ref[...]), 0.0)
            z = jnp.dot(z, mw1_ref[...], preferred_element_type=jnp.float32)
            z = jnp.maximum(bn(z + mb1_ref[...], g1_ref[...], be1_ref[...]), 0.0)
            out_ref[...] = jnp.dot(z, mw2_ref[...],
                                   preferred_element_type=jnp.float32) + mb2_ref[0, 0]

    def full(shape):
        return pl.BlockSpec(shape, lambda i: tuple(0 for _ in shape))

    return pl.pallas_call(
        body,
        grid=(NBQ,),
        in_specs=[
            pl.BlockSpec((BQ, H), lambda i: (i, 0)),
            pl.BlockSpec((BQ, 1), lambda i: (i, 0)),
            full((H, 1)), full((1, 1)),
            full((H, MH)), full((1, MH)), full((1, MH)), full((1, MH)),
            full((MH, MH)), full((1, MH)), full((1, MH)), full((1, MH)),
            full((MH, 1)), full((1, 1)),
        ],
        out_specs=pl.BlockSpec((NG, 1), lambda i: (0, 0)),
        out_shape=jax.ShapeDtypeStruct((NG, 1), jnp.float32),
        scratch_shapes=[pltpu.VMEM((NG, H), jnp.float32)],
    )(h, batch2, ws_w, ws_b, mw0, mb0, g0, be0, mw1, mb1, g1, be1, mw2, mb2)


def kernel(x, edge_index, edge_type, batch,
           rw0, rr0, rb0, rw1, rr1, rb1, rw2, rr2, rb2,
           ws_w, ws_b, mw0, mb0, g0, be0, mw1, mb1, g1, be1, mw2, mb2):
    i32 = jnp.int32
    src = edge_index[0].astype(i32)
    dst = edge_index[1].astype(i32)
    et = edge_type.reshape(-1).astype(i32)

    key = et * N + dst
    key_s, src_s = lax.sort_key_val(key, src)
    newbin = jnp.concatenate([jnp.ones((1,), i32),
                              (key_s[1:] != key_s[:-1]).astype(i32)])
    bidx = jnp.cumsum(newbin).astype(i32) - 1
    barr = jnp.arange(E, dtype=i32)
    first_e = jnp.searchsorted(bidx, barr, side="left").astype(i32)
    last_e = jnp.searchsorted(bidx, barr, side="right").astype(i32) - 1
    validb = first_e < E
    c_b = jnp.where(validb, last_e - first_e + 1, 1)

    startbin = bidx[::BB]
    endbin = bidx[BB - 1::BB]
    lbcol = (bidx - jnp.repeat(startbin, BB)).reshape(E, 1)
    cont_in = jnp.concatenate([
        jnp.zeros((1,), i32), (startbin[1:] == endbin[:-1]).astype(i32)])
    lbmax = endbin - startbin

    key_b = jnp.take(key_s, jnp.clip(first_e, 0, E - 1))
    lbend_b = jnp.take(lbcol.reshape(E), jnp.clip(last_e, 0, E - 1))
    tb = jnp.stack([key_b, c_b, last_e, lbend_b], axis=1)

    key_bm = jnp.where(validb, key_b, BIGKEY)
    bstart = jnp.searchsorted(key_bm, jnp.arange(R + 1, dtype=i32) * N,
                              side="left").astype(i32)
    brelcnt = bstart[1:] - bstart[:-1]
    nblkb = (brelcnt + BB - 1) // BB
    blk_end = jnp.cumsum(nblkb).astype(i32)
    blk2rel = jnp.minimum(
        jnp.searchsorted(blk_end, jnp.arange(NBLK, dtype=i32), side="right"),
        R - 1).astype(i32)
    pad_off = (blk_end - nblkb) * BB

    padoff_q = jnp.repeat(jnp.take(pad_off, blk2rel), BB)
    bstart_q = jnp.repeat(jnp.take(bstart[:-1], blk2rel), BB)
    bc = jnp.repeat(jnp.take(brelcnt, blk2rel), BB)
    qq = jnp.arange(EP, dtype=i32)
    pos = qq - padoff_q
    validq = (pos >= 0) & (pos < bc)
    b_q = jnp.clip(bstart_q + jnp.clip(pos, 0, E - 1), 0, E - 1)
    tq = jnp.take(tb, b_q, axis=0)
    dstq = jnp.where(validq, tq[:, 0] % N, 0)
    cntq = jnp.where(validq, tq[:, 1].astype(jnp.float32), jnp.inf)
    binpos = jnp.where(validq, (tq[:, 2] // BB) * BB + tq[:, 3], 0)

    src_pad = jnp.concatenate([src_s, jnp.zeros((EG - E,), i32)])
    srcp3 = src_pad.reshape(NW, CWE, 128)
    binpos3 = binpos.reshape(NW, CW, 128)
    dstq3 = dstq.reshape(NW, CW, 128)
    cntq2 = cntq.reshape(EP, 1)
    zn = jnp.zeros((128, H), jnp.float32)

    h = x
    for rw, rr, rb in ((rw0, rr0, rb0), (rw1, rr1, rb1), (rw2, rr2, rb2)):
        xs = _gather_sc(h, srcp3, CWE)
        bsum_ends = _binsum_tc(xs, lbcol, cont_in, lbmax)
        bsum = _gather_sc(bsum_ends, binpos3, CW)
        msg = _relmm_tc(bsum, cntq2, rw, blk2rel)
        part = _scatter_sc(msg, dstq3, zn).reshape(NC, NPAD, H)
        h = _combine_tc(part[0, :N], part[1, :N], h, rr, rb.reshape(1, H))

    return _head_tc(h, batch.reshape(N, 1).astype(i32),
                    ws_w, ws_b.reshape(1, 1), mw0, mb0.reshape(1, MH),
                    g0.reshape(1, MH), be0.reshape(1, MH), mw1,
                    mb1.reshape(1, MH), g1.reshape(1, MH), be1.reshape(1, MH),
                    mw2, mb2.reshape(1, 1))
